# Initial kernel scaffold; baseline (speedup 1.0000x reference)
import jax
import jax.numpy as jnp
from jax.experimental import pallas as pl


def kernel(batch_data, item_embeddings):
    raise NotImplementedError("write your pallas kernel here")


# speedup vs baseline: 1.7963x; 1.7963x over previous
import functools

import jax
import jax.numpy as jnp
from jax import lax
from jax.experimental import pallas as pl
from jax.experimental.pallas import tpu as pltpu
from jax.experimental.pallas import tpu_sc as plsc

_D = 64
_NW = 32
_CHUNK = 512


def _make_gather(n_rows):
    per_w = n_rows // _NW
    n_chunks = per_w // _CHUNK
    mesh = plsc.VectorSubcoreMesh(core_axis_name="c", subcore_axis_name="s")

    @functools.partial(
        pl.kernel,
        mesh=mesh,
        out_type=jax.ShapeDtypeStruct((n_rows, _D), jnp.float32),
        compiler_params=pltpu.CompilerParams(use_tc_tiling_on_sc=False),
        scratch_types=[
            pltpu.VMEM((_CHUNK,), jnp.int32),
            pltpu.VMEM((_CHUNK, _D), jnp.float32),
            pltpu.SemaphoreType.DMA,
        ],
    )
    def gather_kernel(table_hbm, idx_hbm, out_hbm, idx_v, rows_v, sem):
        wid = lax.axis_index("s") * 2 + lax.axis_index("c")
        base = wid * per_w

        def body(g, carry):
            off = base + g * _CHUNK
            pltpu.sync_copy(idx_hbm.at[pl.ds(off, _CHUNK)], idx_v)
            pltpu.async_copy(table_hbm.at[idx_v], rows_v, sem).wait()
            pltpu.sync_copy(rows_v, out_hbm.at[pl.ds(off, _CHUNK)])
            return carry

        lax.fori_loop(0, n_chunks, body, 0)

    return gather_kernel


def kernel(batch_data, item_embeddings):
    b, s = batch_data.shape
    idx = batch_data.reshape(-1).astype(jnp.int32)
    out = _make_gather(b * s)(item_embeddings, idx)
    return out.reshape(b, s, _D)


# speedup vs baseline: 1.8745x; 1.0436x over previous
import functools

import jax
import jax.numpy as jnp
from jax import lax
from jax.experimental import pallas as pl
from jax.experimental.pallas import tpu as pltpu
from jax.experimental.pallas import tpu_sc as plsc

_D = 64
_NW = 32
_CHUNK = 640


def _make_gather(n_rows):
    per_w = n_rows // _NW
    n_chunks = per_w // _CHUNK
    assert n_chunks % 2 == 0
    mesh = plsc.VectorSubcoreMesh(core_axis_name="c", subcore_axis_name="s")

    @functools.partial(
        pl.kernel,
        mesh=mesh,
        out_type=jax.ShapeDtypeStruct((n_rows, _D), jnp.float32),
        compiler_params=pltpu.CompilerParams(use_tc_tiling_on_sc=False),
        scratch_types=[
            pltpu.VMEM((per_w,), jnp.int32),
            pltpu.VMEM((_CHUNK, _D), jnp.float32),
            pltpu.VMEM((_CHUNK, _D), jnp.float32),
            pltpu.SemaphoreType.DMA,
            pltpu.SemaphoreType.DMA,
            pltpu.SemaphoreType.DMA,
            pltpu.SemaphoreType.DMA,
        ],
    )
    def gather_kernel(table_hbm, idx_hbm, out_hbm,
                      idx_v, rows0, rows1, gs0, gs1, os0, os1):
        wid = lax.axis_index("s") * 2 + lax.axis_index("c")
        base = wid * per_w
        slots = ((rows0, gs0, os0), (rows1, gs1, os1))

        pltpu.sync_copy(idx_hbm.at[pl.ds(base, per_w)], idx_v)

        def start_gather(g, sl):
            rows, gsem, _ = sl
            pltpu.async_copy(
                table_hbm.at[idx_v.at[pl.ds(g * _CHUNK, _CHUNK)]], rows, gsem)

        def wait_gather(sl):
            rows, gsem, _ = sl
            pltpu.make_async_copy(
                table_hbm.at[idx_v.at[pl.ds(0, _CHUNK)]], rows, gsem).wait()

        def start_out(g, sl):
            rows, _, osem = sl
            pltpu.async_copy(
                rows, out_hbm.at[pl.ds(base + g * _CHUNK, _CHUNK)], osem)

        def wait_out(sl):
            rows, _, osem = sl
            pltpu.make_async_copy(
                rows, out_hbm.at[pl.ds(base, _CHUNK)], osem).wait()

        start_gather(0, slots[0])

        def outer(i, carry):
            g0 = 2 * i
            @pl.when(i > 0)
            def _():
                wait_out(slots[1])
            start_gather(g0 + 1, slots[1])
            wait_gather(slots[0])
            start_out(g0, slots[0])

            @pl.when(g0 + 2 < n_chunks)
            def _():
                wait_out(slots[0])
                start_gather(g0 + 2, slots[0])
            wait_gather(slots[1])
            start_out(g0 + 1, slots[1])
            return carry

        lax.fori_loop(0, n_chunks // 2, outer, 0)
        wait_out(slots[0])
        wait_out(slots[1])

    return gather_kernel


def kernel(batch_data, item_embeddings):
    b, s = batch_data.shape
    idx = batch_data.reshape(-1).astype(jnp.int32)
    out = _make_gather(b * s)(item_embeddings, idx)
    return out.reshape(b, s, _D)
